# Initial kernel scaffold; baseline (speedup 1.0000x reference)
import jax
import jax.numpy as jnp
from jax.experimental import pallas as pl


def kernel(x_encoded, edge_index, mapping_idx, weight, w_ih, w_hh, b_ih, b_hh):
    raise NotImplementedError("write your pallas kernel here")


# speedup vs baseline: 6.5348x; 6.5348x over previous
import functools

import jax
import jax.numpy as jnp
from jax import lax
from jax.experimental import pallas as pl
from jax.experimental.pallas import tpu as pltpu
from jax.experimental.pallas import tpu_sc as plsc

N = 10000
E = 320000
D = 128
L = 3

NC = 2
NS = 16
NW = NC * NS
EPW = E // NW
CHUNK = 80
NCHUNK = EPW // CHUNK
UNIT = 80
NUNITS = N // UNIT

BLK = 1000


def _segment_sum_partials(m, src3, dst3):
    mesh = plsc.VectorSubcoreMesh(core_axis_name="c", subcore_axis_name="s")

    @functools.partial(
        pl.kernel,
        mesh=mesh,
        out_type=jax.ShapeDtypeStruct((NC, N, D), jnp.float32),
        scratch_types=[
            pltpu.VMEM((NCHUNK, CHUNK), jnp.int32),
            pltpu.VMEM((NCHUNK, CHUNK), jnp.int32),
            pltpu.VMEM((CHUNK, D), jnp.float32),
            pltpu.VMEM_SHARED((N, D), jnp.float32),
            pltpu.SemaphoreType.DMA,
        ],
    )
    def seg_kernel(m_hbm, src_hbm, dst_hbm, out_hbm,
                   src_v, dst_v, rows_v, agg_sh, sem):
        c = lax.axis_index("c")
        s = lax.axis_index("s")
        wid = c * NS + s
        nu = jnp.where(s < NUNITS - NS * (NUNITS // NS), NUNITS // NS + 1,
                       NUNITS // NS)

        def zfill(i, carry):
            for g in range(D // 16):
                rows_v[i, pl.ds(g * 16, 16)] = jnp.zeros((16,), jnp.float32)
            return carry

        lax.fori_loop(0, UNIT, zfill, 0)

        def zcopy(k, carry):
            pltpu.sync_copy(rows_v, agg_sh.at[pl.ds((s + NS * k) * UNIT, UNIT)])
            return carry

        lax.fori_loop(0, nu, zcopy, 0)

        pltpu.sync_copy(src_hbm.at[wid], src_v)
        pltpu.sync_copy(dst_hbm.at[wid], dst_v)

        plsc.subcore_barrier()

        def edge_body(j, carry):
            pltpu.async_copy(m_hbm.at[src_v.at[j]], rows_v, sem).wait()
            pltpu.sync_copy(rows_v, agg_sh.at[dst_v.at[j]], add=True)
            return carry

        lax.fori_loop(0, NCHUNK, edge_body, 0)

        plsc.subcore_barrier()

        def wcopy(k, carry):
            off = (s + NS * k) * UNIT
            pltpu.sync_copy(agg_sh.at[pl.ds(off, UNIT)],
                            out_hbm.at[c, pl.ds(off, UNIT)])
            return carry

        lax.fori_loop(0, nu, wcopy, 0)

    return seg_kernel(m, src3, dst3)


def _mm_body(x_ref, w_ref, o_ref):
    o_ref[...] = jnp.dot(x_ref[...], w_ref[...],
                         preferred_element_type=jnp.float32)


def _message_mm(x, w):
    return pl.pallas_call(
        _mm_body,
        grid=(N // BLK,),
        in_specs=[
            pl.BlockSpec((BLK, D), lambda i: (i, 0)),
            pl.BlockSpec((D, D), lambda i: (0, 0)),
        ],
        out_specs=pl.BlockSpec((BLK, D), lambda i: (i, 0)),
        out_shape=jax.ShapeDtypeStruct((N, D), jnp.float32),
    )(x, w)


def _gru_body(p_ref, h_ref, wih_ref, whh_ref, bih_ref, bhh_ref, wn_ref,
              hy_ref, mn_ref):
    agg = p_ref[0] + p_ref[1]
    h = h_ref[...]
    gi = jnp.dot(agg, wih_ref[...], preferred_element_type=jnp.float32)
    gi = gi + bih_ref[...]
    gh = jnp.dot(h, whh_ref[...], preferred_element_type=jnp.float32)
    gh = gh + bhh_ref[...]
    r = jax.nn.sigmoid(gi[:, :D] + gh[:, :D])
    z = jax.nn.sigmoid(gi[:, D:2 * D] + gh[:, D:2 * D])
    n = jnp.tanh(gi[:, 2 * D:] + r * gh[:, 2 * D:])
    hy = (1.0 - z) * n + z * h
    hy_ref[...] = hy
    mn_ref[...] = jnp.dot(hy, wn_ref[...], preferred_element_type=jnp.float32)


def _gru_layer(p, h, wih_t, whh_t, bih, bhh, w_next):
    return pl.pallas_call(
        _gru_body,
        grid=(N // BLK,),
        in_specs=[
            pl.BlockSpec((2, BLK, D), lambda i: (0, i, 0)),
            pl.BlockSpec((BLK, D), lambda i: (i, 0)),
            pl.BlockSpec((D, 3 * D), lambda i: (0, 0)),
            pl.BlockSpec((D, 3 * D), lambda i: (0, 0)),
            pl.BlockSpec((1, 3 * D), lambda i: (0, 0)),
            pl.BlockSpec((1, 3 * D), lambda i: (0, 0)),
            pl.BlockSpec((D, D), lambda i: (0, 0)),
        ],
        out_specs=[
            pl.BlockSpec((BLK, D), lambda i: (i, 0)),
            pl.BlockSpec((BLK, D), lambda i: (i, 0)),
        ],
        out_shape=[
            jax.ShapeDtypeStruct((N, D), jnp.float32),
            jax.ShapeDtypeStruct((N, D), jnp.float32),
        ],
    )(p, h, wih_t, whh_t, bih, bhh, w_next)


def kernel(x_encoded, edge_index, mapping_idx, weight, w_ih, w_hh, b_ih, b_hh):
    del mapping_idx
    src3 = edge_index[0].reshape(NW, NCHUNK, CHUNK)
    dst3 = edge_index[1].reshape(NW, NCHUNK, CHUNK)
    wih_t = w_ih.T
    whh_t = w_hh.T
    bih = b_ih.reshape(1, 3 * D)
    bhh = b_hh.reshape(1, 3 * D)

    h = x_encoded
    m = _message_mm(x_encoded, weight[0])
    for i in range(L):
        p = _segment_sum_partials(m, src3, dst3)
        h, m = _gru_layer(p, h, wih_t, whh_t, bih, bhh, weight[(i + 1) % L])
    return h


# speedup vs baseline: 8.1926x; 1.2537x over previous
import functools

import jax
import jax.numpy as jnp
from jax import lax
from jax.experimental import pallas as pl
from jax.experimental.pallas import tpu as pltpu
from jax.experimental.pallas import tpu_sc as plsc

N = 10000
E = 320000
D = 128
L = 3

NC = 2
NS = 16
NW = NC * NS
EPW = E // NW
CHUNK = 80
NCHUNK = EPW // CHUNK
UNIT = 80
NUNITS = N // UNIT

BLK = 1000


def _segment_sum_partials(m, edges4):
    mesh = plsc.VectorSubcoreMesh(core_axis_name="c", subcore_axis_name="s")

    @functools.partial(
        pl.kernel,
        mesh=mesh,
        out_type=jax.ShapeDtypeStruct((NC, N, D), jnp.float32),
        scratch_types=[
            pltpu.VMEM((3, 2, CHUNK), jnp.int32),
            pltpu.VMEM((2, CHUNK, D), jnp.float32),
            pltpu.VMEM_SHARED((N, D), jnp.float32),
            pltpu.SemaphoreType.DMA,
            pltpu.SemaphoreType.DMA,
            pltpu.SemaphoreType.DMA,
        ],
    )
    def seg_kernel(m_hbm, edges_hbm, out_hbm,
                   idx_v, rows_v, agg_sh, isem, gsem, ssem):
        c = lax.axis_index("c")
        s = lax.axis_index("s")
        wid = c * NS + s
        nu = jnp.where(s < NUNITS - NS * (NUNITS // NS), NUNITS // NS + 1,
                       NUNITS // NS)

        def zfill(i, carry):
            for g in range(D // 16):
                rows_v[0, i, pl.ds(g * 16, 16)] = jnp.zeros((16,), jnp.float32)
            return carry

        lax.fori_loop(0, UNIT, zfill, 0)

        def zcopy(k, carry):
            pltpu.sync_copy(rows_v.at[0],
                            agg_sh.at[pl.ds((s + NS * k) * UNIT, UNIT)])
            return carry

        lax.fori_loop(0, nu, zcopy, 0)

        plsc.subcore_barrier()

        pltpu.async_copy(edges_hbm.at[wid, 0], idx_v.at[0], isem)
        pltpu.make_async_copy(edges_hbm.at[wid, 0], idx_v.at[0], isem).wait()
        pltpu.async_copy(edges_hbm.at[wid, 1], idx_v.at[1], isem)
        pltpu.async_copy(m_hbm.at[idx_v.at[0, 0]], rows_v.at[0], gsem)

        def edge_body(j, carry):
            b2 = j % 2
            b3 = j % 3
            pltpu.make_async_copy(m_hbm.at[idx_v.at[b3, 0]],
                                  rows_v.at[b2], gsem).wait()

            @pl.when(j >= 1)
            def _():
                pltpu.make_async_copy(
                    rows_v.at[1 - b2],
                    agg_sh.at[idx_v.at[(j - 1) % 3, 1]], ssem).wait()

            @pl.when(j + 1 < NCHUNK)
            def _():
                nb3 = (j + 1) % 3
                pltpu.make_async_copy(edges_hbm.at[wid, j + 1],
                                      idx_v.at[nb3], isem).wait()
                pltpu.async_copy(m_hbm.at[idx_v.at[nb3, 0]],
                                 rows_v.at[1 - b2], gsem)

            @pl.when(j + 2 < NCHUNK)
            def _():
                pltpu.async_copy(edges_hbm.at[wid, j + 2],
                                 idx_v.at[(j + 2) % 3], isem)

            pltpu.async_copy(rows_v.at[b2], agg_sh.at[idx_v.at[b3, 1]],
                             ssem, add=True)
            return carry

        lax.fori_loop(0, NCHUNK, edge_body, 0)
        pltpu.make_async_copy(rows_v.at[(NCHUNK - 1) % 2],
                              agg_sh.at[idx_v.at[(NCHUNK - 1) % 3, 1]],
                              ssem).wait()

        plsc.subcore_barrier()

        def wcopy(k, carry):
            off = (s + NS * k) * UNIT
            pltpu.sync_copy(agg_sh.at[pl.ds(off, UNIT)],
                            out_hbm.at[c, pl.ds(off, UNIT)])
            return carry

        lax.fori_loop(0, nu, wcopy, 0)

    return seg_kernel(m, edges4)


def _mm_body(x_ref, w_ref, o_ref):
    o_ref[...] = jnp.dot(x_ref[...], w_ref[...],
                         preferred_element_type=jnp.float32)


def _message_mm(x, w):
    return pl.pallas_call(
        _mm_body,
        grid=(N // BLK,),
        in_specs=[
            pl.BlockSpec((BLK, D), lambda i: (i, 0)),
            pl.BlockSpec((D, D), lambda i: (0, 0)),
        ],
        out_specs=pl.BlockSpec((BLK, D), lambda i: (i, 0)),
        out_shape=jax.ShapeDtypeStruct((N, D), jnp.float32),
    )(x, w)


def _gru_body(p_ref, h_ref, wih_ref, whh_ref, bih_ref, bhh_ref, wn_ref,
              hy_ref, mn_ref):
    agg = p_ref[0] + p_ref[1]
    h = h_ref[...]
    gi = jnp.dot(agg, wih_ref[...], preferred_element_type=jnp.float32)
    gi = gi + bih_ref[...]
    gh = jnp.dot(h, whh_ref[...], preferred_element_type=jnp.float32)
    gh = gh + bhh_ref[...]
    r = jax.nn.sigmoid(gi[:, :D] + gh[:, :D])
    z = jax.nn.sigmoid(gi[:, D:2 * D] + gh[:, D:2 * D])
    n = jnp.tanh(gi[:, 2 * D:] + r * gh[:, 2 * D:])
    hy = (1.0 - z) * n + z * h
    hy_ref[...] = hy
    mn_ref[...] = jnp.dot(hy, wn_ref[...], preferred_element_type=jnp.float32)


def _gru_layer(p, h, wih_t, whh_t, bih, bhh, w_next):
    return pl.pallas_call(
        _gru_body,
        grid=(N // BLK,),
        in_specs=[
            pl.BlockSpec((2, BLK, D), lambda i: (0, i, 0)),
            pl.BlockSpec((BLK, D), lambda i: (i, 0)),
            pl.BlockSpec((D, 3 * D), lambda i: (0, 0)),
            pl.BlockSpec((D, 3 * D), lambda i: (0, 0)),
            pl.BlockSpec((1, 3 * D), lambda i: (0, 0)),
            pl.BlockSpec((1, 3 * D), lambda i: (0, 0)),
            pl.BlockSpec((D, D), lambda i: (0, 0)),
        ],
        out_specs=[
            pl.BlockSpec((BLK, D), lambda i: (i, 0)),
            pl.BlockSpec((BLK, D), lambda i: (i, 0)),
        ],
        out_shape=[
            jax.ShapeDtypeStruct((N, D), jnp.float32),
            jax.ShapeDtypeStruct((N, D), jnp.float32),
        ],
    )(p, h, wih_t, whh_t, bih, bhh, w_next)


def kernel(x_encoded, edge_index, mapping_idx, weight, w_ih, w_hh, b_ih, b_hh):
    del mapping_idx
    edges4 = jnp.stack([edge_index[0].reshape(NW, NCHUNK, CHUNK),
                        edge_index[1].reshape(NW, NCHUNK, CHUNK)], axis=2)
    wih_t = w_ih.T
    whh_t = w_hh.T
    bih = b_ih.reshape(1, 3 * D)
    bhh = b_hh.reshape(1, 3 * D)

    h = x_encoded
    m = _message_mm(x_encoded, weight[0])
    for i in range(L):
        p = _segment_sum_partials(m, edges4)
        h, m = _gru_layer(p, h, wih_t, whh_t, bih, bhh, weight[(i + 1) % L])
    return h


# speedup vs baseline: 9.7097x; 1.1852x over previous
import functools

import jax
import jax.numpy as jnp
from jax import lax
from jax.experimental import pallas as pl
from jax.experimental.pallas import tpu as pltpu
from jax.experimental.pallas import tpu_sc as plsc

N = 10000
E = 320000
D = 128
L = 3

NC = 2
NS = 16
NW = NC * NS
EPW = E // NW
CHUNK = 125
NCHUNK = EPW // CHUNK
UNIT = 80
NUNITS = N // UNIT

BLK = 1000


def _segment_sum_partials(m, edges4):
    mesh = plsc.VectorSubcoreMesh(core_axis_name="c", subcore_axis_name="s")

    @functools.partial(
        pl.kernel,
        mesh=mesh,
        out_type=jax.ShapeDtypeStruct((NC, N, D), jnp.float32),
        scratch_types=[
            pltpu.VMEM((3, 2, CHUNK), jnp.int32),
            pltpu.VMEM((2, CHUNK, D), jnp.float32),
            pltpu.VMEM_SHARED((N, D), jnp.float32),
            pltpu.SemaphoreType.DMA,
            pltpu.SemaphoreType.DMA,
            pltpu.SemaphoreType.DMA,
        ],
    )
    def seg_kernel(m_hbm, edges_hbm, out_hbm,
                   idx_v, rows_v, agg_sh, isem, gsem, ssem):
        c = lax.axis_index("c")
        s = lax.axis_index("s")
        wid = c * NS + s
        nu = jnp.where(s < NUNITS - NS * (NUNITS // NS), NUNITS // NS + 1,
                       NUNITS // NS)

        def zfill(i, carry):
            for g in range(D // 16):
                rows_v[0, i, pl.ds(g * 16, 16)] = jnp.zeros((16,), jnp.float32)
            return carry

        lax.fori_loop(0, UNIT, zfill, 0)

        def zcopy(k, carry):
            pltpu.sync_copy(rows_v.at[0, pl.ds(0, UNIT)],
                            agg_sh.at[pl.ds((s + NS * k) * UNIT, UNIT)])
            return carry

        lax.fori_loop(0, nu, zcopy, 0)

        plsc.subcore_barrier()

        pltpu.async_copy(edges_hbm.at[wid, 0], idx_v.at[0], isem)
        pltpu.make_async_copy(edges_hbm.at[wid, 0], idx_v.at[0], isem).wait()
        pltpu.async_copy(edges_hbm.at[wid, 1], idx_v.at[1], isem)
        pltpu.async_copy(m_hbm.at[idx_v.at[0, 0]], rows_v.at[0], gsem)

        def edge_body(j, carry):
            b2 = j % 2
            b3 = j % 3
            pltpu.make_async_copy(m_hbm.at[idx_v.at[b3, 0]],
                                  rows_v.at[b2], gsem).wait()

            @pl.when(j >= 1)
            def _():
                pltpu.make_async_copy(
                    rows_v.at[1 - b2],
                    agg_sh.at[idx_v.at[(j - 1) % 3, 1]], ssem).wait()

            @pl.when(j + 1 < NCHUNK)
            def _():
                nb3 = (j + 1) % 3
                pltpu.make_async_copy(edges_hbm.at[wid, j + 1],
                                      idx_v.at[nb3], isem).wait()
                pltpu.async_copy(m_hbm.at[idx_v.at[nb3, 0]],
                                 rows_v.at[1 - b2], gsem)

            @pl.when(j + 2 < NCHUNK)
            def _():
                pltpu.async_copy(edges_hbm.at[wid, j + 2],
                                 idx_v.at[(j + 2) % 3], isem)

            pltpu.async_copy(rows_v.at[b2], agg_sh.at[idx_v.at[b3, 1]],
                             ssem, add=True)
            return carry

        lax.fori_loop(0, NCHUNK, edge_body, 0)
        pltpu.make_async_copy(rows_v.at[(NCHUNK - 1) % 2],
                              agg_sh.at[idx_v.at[(NCHUNK - 1) % 3, 1]],
                              ssem).wait()

        plsc.subcore_barrier()

        def wcopy(k, carry):
            off = (s + NS * k) * UNIT
            pltpu.sync_copy(agg_sh.at[pl.ds(off, UNIT)],
                            out_hbm.at[c, pl.ds(off, UNIT)])
            return carry

        lax.fori_loop(0, nu, wcopy, 0)

    return seg_kernel(m, edges4)


def _mm_body(x_ref, w_ref, o_ref):
    o_ref[...] = jnp.dot(x_ref[...], w_ref[...],
                         preferred_element_type=jnp.float32)


def _message_mm(x, w):
    return pl.pallas_call(
        _mm_body,
        grid=(N // BLK,),
        in_specs=[
            pl.BlockSpec((BLK, D), lambda i: (i, 0)),
            pl.BlockSpec((D, D), lambda i: (0, 0)),
        ],
        out_specs=pl.BlockSpec((BLK, D), lambda i: (i, 0)),
        out_shape=jax.ShapeDtypeStruct((N, D), jnp.float32),
    )(x, w)


def _gru_body(p_ref, h_ref, wih_ref, whh_ref, bih_ref, bhh_ref, wn_ref,
              hy_ref, mn_ref):
    agg = p_ref[0] + p_ref[1]
    h = h_ref[...]
    gi = jnp.dot(agg, wih_ref[...], preferred_element_type=jnp.float32)
    gi = gi + bih_ref[...]
    gh = jnp.dot(h, whh_ref[...], preferred_element_type=jnp.float32)
    gh = gh + bhh_ref[...]
    r = jax.nn.sigmoid(gi[:, :D] + gh[:, :D])
    z = jax.nn.sigmoid(gi[:, D:2 * D] + gh[:, D:2 * D])
    n = jnp.tanh(gi[:, 2 * D:] + r * gh[:, 2 * D:])
    hy = (1.0 - z) * n + z * h
    hy_ref[...] = hy
    mn_ref[...] = jnp.dot(hy, wn_ref[...], preferred_element_type=jnp.float32)


def _gru_layer(p, h, wih_t, whh_t, bih, bhh, w_next):
    return pl.pallas_call(
        _gru_body,
        grid=(N // BLK,),
        in_specs=[
            pl.BlockSpec((2, BLK, D), lambda i: (0, i, 0)),
            pl.BlockSpec((BLK, D), lambda i: (i, 0)),
            pl.BlockSpec((D, 3 * D), lambda i: (0, 0)),
            pl.BlockSpec((D, 3 * D), lambda i: (0, 0)),
            pl.BlockSpec((1, 3 * D), lambda i: (0, 0)),
            pl.BlockSpec((1, 3 * D), lambda i: (0, 0)),
            pl.BlockSpec((D, D), lambda i: (0, 0)),
        ],
        out_specs=[
            pl.BlockSpec((BLK, D), lambda i: (i, 0)),
            pl.BlockSpec((BLK, D), lambda i: (i, 0)),
        ],
        out_shape=[
            jax.ShapeDtypeStruct((N, D), jnp.float32),
            jax.ShapeDtypeStruct((N, D), jnp.float32),
        ],
    )(p, h, wih_t, whh_t, bih, bhh, w_next)


def kernel(x_encoded, edge_index, mapping_idx, weight, w_ih, w_hh, b_ih, b_hh):
    del mapping_idx
    edges4 = jnp.stack([edge_index[0].reshape(NW, NCHUNK, CHUNK),
                        edge_index[1].reshape(NW, NCHUNK, CHUNK)], axis=2)
    wih_t = w_ih.T
    whh_t = w_hh.T
    bih = b_ih.reshape(1, 3 * D)
    bhh = b_hh.reshape(1, 3 * D)

    h = x_encoded
    m = _message_mm(x_encoded, weight[0])
    for i in range(L):
        p = _segment_sum_partials(m, edges4)
        h, m = _gru_layer(p, h, wih_t, whh_t, bih, bhh, weight[(i + 1) % L])
    return h


# speedup vs baseline: 11.7263x; 1.2077x over previous
import functools

import jax
import jax.numpy as jnp
from jax import lax
from jax.experimental import pallas as pl
from jax.experimental.pallas import tpu as pltpu
from jax.experimental.pallas import tpu_sc as plsc

N = 10000
E = 320000
D = 128
L = 3

NC = 2
NS = 16
NW = NC * NS
EPW = E // NW
CHUNK = 80
NCHUNK = EPW // CHUNK
NRB = 3
NIB = 6
UNIT = 80
NUNITS = N // UNIT

BLK = 1000


def _segment_sum_partials(m, edges4):
    mesh = plsc.VectorSubcoreMesh(core_axis_name="c", subcore_axis_name="s")

    @functools.partial(
        pl.kernel,
        mesh=mesh,
        out_type=jax.ShapeDtypeStruct((NC, N, D), jnp.float32),
        scratch_types=[
            pltpu.VMEM((NIB, 2, CHUNK), jnp.int32),
            pltpu.VMEM((NRB, CHUNK, D), jnp.float32),
            pltpu.VMEM_SHARED((N, D), jnp.float32),
            [pltpu.SemaphoreType.DMA] * NIB,
            [pltpu.SemaphoreType.DMA] * NRB,
            [pltpu.SemaphoreType.DMA] * NRB,
        ],
    )
    def seg_kernel(m_hbm, edges_hbm, out_hbm,
                   idx_v, rows_v, agg_sh, isems, gsems, ssems):
        c = lax.axis_index("c")
        s = lax.axis_index("s")
        wid = c * NS + s
        nu = jnp.where(s < NUNITS - NS * (NUNITS // NS), NUNITS // NS + 1,
                       NUNITS // NS)

        def zfill(i, carry):
            for g in range(D // 16):
                rows_v[0, i, pl.ds(g * 16, 16)] = jnp.zeros((16,), jnp.float32)
            return carry

        lax.fori_loop(0, UNIT, zfill, 0)

        def zcopy(k, carry):
            pltpu.sync_copy(rows_v.at[0, pl.ds(0, UNIT)],
                            agg_sh.at[pl.ds((s + NS * k) * UNIT, UNIT)])
            return carry

        lax.fori_loop(0, nu, zcopy, 0)

        plsc.subcore_barrier()

        def idx_load(j, q):
            pltpu.async_copy(edges_hbm.at[wid, j], idx_v.at[q], isems[q])

        def idx_wait(j, q):
            pltpu.make_async_copy(edges_hbm.at[wid, j], idx_v.at[q],
                                  isems[q]).wait()

        def gather(q, r):
            pltpu.async_copy(m_hbm.at[idx_v.at[q, 0]], rows_v.at[r],
                             gsems[r])

        def gather_wait(q, r):
            pltpu.make_async_copy(m_hbm.at[idx_v.at[q, 0]], rows_v.at[r],
                                  gsems[r]).wait()

        def scatter(q, r):
            pltpu.async_copy(rows_v.at[r], agg_sh.at[idx_v.at[q, 1]],
                             ssems[r], add=True)

        def scatter_wait(q, r):
            pltpu.make_async_copy(rows_v.at[r], agg_sh.at[idx_v.at[q, 1]],
                                  ssems[r]).wait()

        def step(j, q, r, first=False):
            gather_wait(q, r)
            scatter(q, r)
            if not first:
                scatter_wait((q + NIB - 1) % NIB, (r + NRB - 1) % NRB)

            @pl.when(j + 2 < NCHUNK)
            def _():
                idx_wait(j + 2, (q + 2) % NIB)
                gather((q + 2) % NIB, (r + 2) % NRB)

            @pl.when(j + 4 < NCHUNK)
            def _():
                idx_load(j + 4, (q + 4) % NIB)

        for q in range(4):
            idx_load(q, q)
        idx_wait(0, 0)
        gather(0, 0)
        idx_wait(1, 1)
        gather(1, 1)

        step(0, 0, 0, first=True)
        step(1, 1, 1)

        def main_body(k, carry):
            j0 = 2 + 6 * k
            for i in range(6):
                step(j0 + i, (2 + i) % NIB, (2 + i) % NRB)
            return carry

        n_main = (NCHUNK - 2 - 3) // 6
        lax.fori_loop(0, n_main, main_body, 0)
        for j in range(2 + 6 * n_main, NCHUNK):
            step(j, j % NIB, j % NRB)

        scatter_wait((NCHUNK - 1) % NIB, (NCHUNK - 1) % NRB)

        plsc.subcore_barrier()

        def wcopy(k, carry):
            off = (s + NS * k) * UNIT
            pltpu.sync_copy(agg_sh.at[pl.ds(off, UNIT)],
                            out_hbm.at[c, pl.ds(off, UNIT)])
            return carry

        lax.fori_loop(0, nu, wcopy, 0)

    return seg_kernel(m, edges4)


def _mm_body(x_ref, w_ref, o_ref):
    o_ref[...] = jnp.dot(x_ref[...], w_ref[...],
                         preferred_element_type=jnp.float32)


def _message_mm(x, w):
    return pl.pallas_call(
        _mm_body,
        grid=(N // BLK,),
        in_specs=[
            pl.BlockSpec((BLK, D), lambda i: (i, 0)),
            pl.BlockSpec((D, D), lambda i: (0, 0)),
        ],
        out_specs=pl.BlockSpec((BLK, D), lambda i: (i, 0)),
        out_shape=jax.ShapeDtypeStruct((N, D), jnp.float32),
    )(x, w)


def _gru_body(p_ref, h_ref, wih_ref, whh_ref, bih_ref, bhh_ref, wn_ref,
              hy_ref, mn_ref):
    agg = p_ref[0] + p_ref[1]
    h = h_ref[...]
    gi = jnp.dot(agg, wih_ref[...], preferred_element_type=jnp.float32)
    gi = gi + bih_ref[...]
    gh = jnp.dot(h, whh_ref[...], preferred_element_type=jnp.float32)
    gh = gh + bhh_ref[...]
    r = jax.nn.sigmoid(gi[:, :D] + gh[:, :D])
    z = jax.nn.sigmoid(gi[:, D:2 * D] + gh[:, D:2 * D])
    n = jnp.tanh(gi[:, 2 * D:] + r * gh[:, 2 * D:])
    hy = (1.0 - z) * n + z * h
    hy_ref[...] = hy
    mn_ref[...] = jnp.dot(hy, wn_ref[...], preferred_element_type=jnp.float32)


def _gru_layer(p, h, wih_t, whh_t, bih, bhh, w_next):
    return pl.pallas_call(
        _gru_body,
        grid=(N // BLK,),
        in_specs=[
            pl.BlockSpec((2, BLK, D), lambda i: (0, i, 0)),
            pl.BlockSpec((BLK, D), lambda i: (i, 0)),
            pl.BlockSpec((D, 3 * D), lambda i: (0, 0)),
            pl.BlockSpec((D, 3 * D), lambda i: (0, 0)),
            pl.BlockSpec((1, 3 * D), lambda i: (0, 0)),
            pl.BlockSpec((1, 3 * D), lambda i: (0, 0)),
            pl.BlockSpec((D, D), lambda i: (0, 0)),
        ],
        out_specs=[
            pl.BlockSpec((BLK, D), lambda i: (i, 0)),
            pl.BlockSpec((BLK, D), lambda i: (i, 0)),
        ],
        out_shape=[
            jax.ShapeDtypeStruct((N, D), jnp.float32),
            jax.ShapeDtypeStruct((N, D), jnp.float32),
        ],
    )(p, h, wih_t, whh_t, bih, bhh, w_next)


def kernel(x_encoded, edge_index, mapping_idx, weight, w_ih, w_hh, b_ih, b_hh):
    del mapping_idx
    edges4 = jnp.stack([edge_index[0].reshape(NW, NCHUNK, CHUNK),
                        edge_index[1].reshape(NW, NCHUNK, CHUNK)], axis=2)
    wih_t = w_ih.T
    whh_t = w_hh.T
    bih = b_ih.reshape(1, 3 * D)
    bhh = b_hh.reshape(1, 3 * D)

    h = x_encoded
    m = _message_mm(x_encoded, weight[0])
    for i in range(L):
        p = _segment_sum_partials(m, edges4)
        h, m = _gru_layer(p, h, wih_t, whh_t, bih, bhh, weight[(i + 1) % L])
    return h
